# baseline (speedup 1.0000x reference)
import functools

import jax
import jax.numpy as jnp
from jax import lax
from jax.experimental import pallas as pl
from jax.experimental.pallas import tpu as pltpu
from jax.experimental.pallas import tpu_sc as plsc

N_LAYERS = 32
N_HEADS = 32
BSZ = 4096
K = 256
TOTAL = N_LAYERS * N_HEADS
W_KQV = 3 * TOTAL

_SCALE = 16777216.0
_BITS = 24


def _row_threshold(m, k):
    rows = m.shape[0]
    lo = jnp.full((rows, 1), 0x3F800000, dtype=jnp.int32)
    hi = jnp.full((rows, 1), 0x40000000, dtype=jnp.int32)

    def body(_, carry):
        lo, hi = carry
        mid = (lo + hi) >> 1
        cnt = jnp.sum((m >= mid).astype(jnp.float32), axis=1, keepdims=True)
        pick = cnt >= float(k)
        lo = jnp.where(pick, mid, lo)
        hi = jnp.where(pick, hi, mid)
        return lo, hi

    lo, hi = jax.lax.fori_loop(0, 23, body, (lo, hi))
    return lo


def _attn_body(rand_ref, noise_ref, perturb_ref, out_ref):
    v = rand_ref[...]
    m = jax.lax.bitcast_convert_type(v + 1.0, jnp.int32)
    thr = _row_threshold(m, K)
    sel = m >= thr
    out_ref[...] = jnp.where(sel, noise_ref[...] + perturb_ref[...], 1.0)


def _kqv_blend_body(rand_ref, thr_ref, nk_ref, nq_ref, nv_ref,
                    pk_ref, pq_ref, pv_ref, ok_ref, oq_ref, ov_ref):
    v = rand_ref[...]
    sel = v >= thr_ref[...]
    ok_ref[...] = jnp.where(sel[:, 0:TOTAL], nk_ref[...] + pk_ref[...], 1.0)
    oq_ref[...] = jnp.where(sel[:, TOTAL:2 * TOTAL], nq_ref[...] + pq_ref[...], 1.0)
    ov_ref[...] = jnp.where(sel[:, 2 * TOTAL:3 * TOTAL], nv_ref[...] + pv_ref[...], 1.0)


_NC = 2
_NS = 16
_NW = _NC * _NS
_ROWS_PER_W = BSZ // _NW
_BLK = 8
_CHUNKS = W_KQV // 16


_NBLK = _ROWS_PER_W // _BLK
_KSCALE = 16777216.0


def _scan16(c, target, base_count):
    mask = (c + base_count) < target
    below = plsc.all_reduce_population_count(mask)[0]
    c_above = jnp.max(jnp.where(mask, c + base_count, 0))
    return below, c_above


def _sc_kqv_body(rand_hbm, out_hbm, stage, hist, cand_a, cand_b, thr_buf,
                 sem0, sem1):
    wid = lax.axis_index("s") * _NC + lax.axis_index("c")
    base = wid * _ROWS_PER_W
    lane = lax.iota(jnp.int32, 16)
    ones16 = jnp.ones((16,), jnp.int32)
    zeros16 = jnp.zeros((16,), jnp.int32)

    def refine(shift, src, dst, cnt, j):
        def do_refine(cnt, j):
            hist[pl.ds(0, 16)] = zeros16
            nch = (cnt + 15) // 16

            def scat(i, _):
                v = src[pl.ds(i * 16, 16)]
                valid = lane < (cnt - i * 16)
                ku = plsc.bitcast(v + 1.0, jnp.int32)
                u = 15 - ((ku >> shift) & 15)
                plsc.addupdate_scatter(hist, [u], ones16, mask=valid)
                return 0
            lax.fori_loop(0, nch, scat, 0)

            c = plsc.cumsum(hist[pl.ds(0, 16)])
            below, c_above = _scan16(c, j, jnp.int32(0))
            sub = 15 - below
            j2 = j - c_above

            def comp(i, w):
                v = src[pl.ds(i * 16, 16)]
                valid = lane < (cnt - i * 16)
                ku = plsc.bitcast(v + 1.0, jnp.int32)
                u = (ku >> shift) & 15
                m = valid & (u == sub)
                plsc.store_compressed(dst.at[pl.ds(w, 16)], v, mask=m)
                return w + plsc.all_reduce_population_count(m)[0]
            cnt2 = lax.fori_loop(0, nch, comp, jnp.int32(0))
            return cnt2, j2

        def passthrough(cnt, j):
            dst[pl.ds(0, 16)] = src[pl.ds(0, 16)]
            return cnt, j

        return lax.cond(cnt > 16, do_refine, passthrough, cnt, j)

    def row_body(row_idx, buf, r):
        for i in range(16):
            hist[pl.ds(i * 16, 16)] = zeros16

        @plsc.parallel_loop(0, _CHUNKS, unroll=16)
        def _(i):
            v = stage[buf, r, pl.ds(i * 16, 16)]
            ku = plsc.bitcast(v + 1.0, jnp.int32)
            du = jnp.maximum((jnp.int32(0x3FFFFFFF) - ku) >> 15, 0)
            plsc.addupdate_scatter(hist, [du], ones16)

        def scan(i, carry):
            run, b_desc, c_above = carry
            c = plsc.cumsum(hist[pl.ds(i * 16, 16)])
            below, ca = _scan16(c, jnp.int32(K), run)
            return run + c[15], b_desc + below, jnp.maximum(c_above, ca)

        zero = jnp.int32(0)
        _, b_desc, c_above = lax.fori_loop(0, 16, scan, (zero, zero, zero),
                                           unroll=4)
        j = jnp.int32(K) - c_above

        def comp(i, w):
            v = stage[buf, r, pl.ds(i * 16, 16)]
            ku = plsc.bitcast(v + 1.0, jnp.int32)
            du = jnp.maximum((jnp.int32(0x3FFFFFFF) - ku) >> 15, 0)
            m = du == b_desc
            plsc.store_compressed(cand_a.at[pl.ds(w, 16)], v, mask=m)
            return w + plsc.all_reduce_population_count(m)[0]
        cnt = plsc.parallel_loop(0, _CHUNKS, unroll=16,
                                 carry=jnp.int32(0))(comp)

        cnt, j = refine(11, cand_a, cand_b, cnt, j)
        cnt, j = refine(7, cand_b, cand_a, cnt, j)

        v = cand_a[pl.ds(0, 16)]
        v = jnp.where(lane < cnt, v, -1.0)
        sv = plsc.sort_key_val(v, v, descending=True)[0]
        jc = jnp.minimum(j, jnp.minimum(cnt, 16))
        thr = jnp.max(jnp.where(lane == jc - 1, sv, -1.0))

        idx = jnp.full((16,), row_idx, jnp.int32)
        plsc.store_scatter(thr_buf, [idx, jnp.zeros((16,), jnp.int32)],
                           jnp.full((16,), thr, jnp.float32), mask=lane == 0)
        return 0

    sems = (sem0, sem1)

    def start_copy(b, buf):
        pltpu.async_copy(rand_hbm.at[pl.ds(base + b * _BLK, _BLK)],
                         stage.at[buf], sems[buf])

    def wait_copy(b, buf):
        pltpu.make_async_copy(rand_hbm.at[pl.ds(base + b * _BLK, _BLK)],
                              stage.at[buf], sems[buf]).wait()

    def process_blk(b, buf):
        def inner(r, _):
            return row_body(b * _BLK + r, buf, r)
        lax.fori_loop(0, _BLK, inner, 0)

    start_copy(0, 0)

    def blk_pair(p, _):
        b0 = p * 2
        start_copy(b0 + 1, 1)
        wait_copy(b0, 0)
        process_blk(b0, 0)

        @pl.when(b0 + 2 < _NBLK)
        def _():
            start_copy(b0 + 2, 0)

        wait_copy(b0 + 1, 1)
        process_blk(b0 + 1, 1)
        return 0

    lax.fori_loop(0, _NBLK // 2, blk_pair, 0)
    pltpu.sync_copy(thr_buf, out_hbm.at[pl.ds(base, _ROWS_PER_W)])


@functools.partial(
    pl.kernel,
    out_type=jax.ShapeDtypeStruct((BSZ, 1), jnp.float32),
    mesh=plsc.VectorSubcoreMesh(core_axis_name="c", subcore_axis_name="s"),
    scratch_types=[
        pltpu.VMEM((2, _BLK, W_KQV), jnp.float32),
        pltpu.VMEM((256,), jnp.int32),
        pltpu.VMEM((W_KQV + 16,), jnp.float32),
        pltpu.VMEM((W_KQV + 16,), jnp.float32),
        pltpu.VMEM((_ROWS_PER_W, 1), jnp.float32),
        pltpu.SemaphoreType.DMA,
        pltpu.SemaphoreType.DMA,
    ],
    compiler_params=pltpu.CompilerParams(needs_layout_passes=False),
)
def _sc_kqv_thresholds(rand_hbm, out_hbm, stage, hist, cand_a, cand_b,
                       thr_buf, sem0, sem1):
    _sc_kqv_body(rand_hbm, out_hbm, stage, hist, cand_a, cand_b,
                 thr_buf, sem0, sem1)


def kernel(rand_attn, noise_attn, rand_kqv, noise_k, noise_q, noise_v,
           perturb_attn, perturb_k, perturb_q, perturb_v):
    R = 256
    grid = (BSZ // R,)

    row_spec = pl.BlockSpec((R, TOTAL), lambda i: (i, 0))
    kqv_spec = pl.BlockSpec((R, W_KQV), lambda i: (i, 0))
    p_spec = pl.BlockSpec((1, TOTAL), lambda i: (0, 0))
    thr_spec = pl.BlockSpec((R, 1), lambda i: (i, 0))
    out3_spec = pl.BlockSpec((R, N_LAYERS, N_HEADS), lambda i: (i, 0, 0))
    out3_shape = jax.ShapeDtypeStruct((BSZ, N_LAYERS, N_HEADS), jnp.float32)

    attn_mask = pl.pallas_call(
        _attn_body,
        grid=grid,
        in_specs=[row_spec, row_spec, p_spec],
        out_specs=row_spec,
        out_shape=jax.ShapeDtypeStruct((BSZ, TOTAL), jnp.float32),
    )(rand_attn, noise_attn, perturb_attn.reshape(1, TOTAL))

    thr_kqv = _sc_kqv_thresholds(rand_kqv)

    k_mask, q_mask, v_mask = pl.pallas_call(
        _kqv_blend_body,
        grid=grid,
        in_specs=[kqv_spec, thr_spec, row_spec, row_spec, row_spec,
                  p_spec, p_spec, p_spec],
        out_specs=[row_spec, row_spec, row_spec],
        out_shape=[jax.ShapeDtypeStruct((BSZ, TOTAL), jnp.float32)] * 3,
    )(rand_kqv, thr_kqv, noise_k, noise_q, noise_v,
      perturb_k.reshape(1, TOTAL), perturb_q.reshape(1, TOTAL),
      perturb_v.reshape(1, TOTAL))

    shape = (BSZ, N_LAYERS, N_HEADS)
    return (attn_mask.reshape(shape), k_mask.reshape(shape),
            q_mask.reshape(shape), v_mask.reshape(shape))


# speedup vs baseline: 1.1098x; 1.1098x over previous
import functools

import jax
import jax.numpy as jnp
from jax import lax
from jax.experimental import pallas as pl
from jax.experimental.pallas import tpu as pltpu
from jax.experimental.pallas import tpu_sc as plsc

N_LAYERS = 32
N_HEADS = 32
BSZ = 4096
K = 256
TOTAL = N_LAYERS * N_HEADS
W_KQV = 3 * TOTAL

_SCALE = 16777216.0
_BITS = 24


def _row_threshold(m, k):
    rows = m.shape[0]
    lo = jnp.full((rows, 1), 0x3F800000, dtype=jnp.int32)
    hi = jnp.full((rows, 1), 0x40000000, dtype=jnp.int32)

    def body(_, carry):
        lo, hi = carry
        mid = (lo + hi) >> 1
        cnt = jnp.sum((m >= mid).astype(jnp.float32), axis=1, keepdims=True)
        pick = cnt >= float(k)
        lo = jnp.where(pick, mid, lo)
        hi = jnp.where(pick, hi, mid)
        return lo, hi

    lo, hi = jax.lax.fori_loop(0, 23, body, (lo, hi))
    return lo


def _attn_body(rand_ref, noise_ref, perturb_ref, out_ref):
    v = rand_ref[...]
    m = jax.lax.bitcast_convert_type(v + 1.0, jnp.int32)
    thr = _row_threshold(m, K)
    sel = m >= thr
    out_ref[...] = jnp.where(sel, noise_ref[...] + perturb_ref[...], 1.0)


def _kqv_blend_body(rand_ref, thr_ref, nk_ref, nq_ref, nv_ref,
                    pk_ref, pq_ref, pv_ref, ok_ref, oq_ref, ov_ref):
    v = rand_ref[...]
    sel = v >= thr_ref[...]
    ok_ref[...] = jnp.where(sel[:, 0:TOTAL], nk_ref[...] + pk_ref[...], 1.0)
    oq_ref[...] = jnp.where(sel[:, TOTAL:2 * TOTAL], nq_ref[...] + pq_ref[...], 1.0)
    ov_ref[...] = jnp.where(sel[:, 2 * TOTAL:3 * TOTAL], nv_ref[...] + pv_ref[...], 1.0)


_NC = 2
_NS = 16
_NW = _NC * _NS
_ROWS_PER_W = BSZ // _NW
_BLK = 8
_CHUNKS = W_KQV // 16


_NBLK = _ROWS_PER_W // _BLK
_KSCALE = 16777216.0


def _scan16(c, target, base_count):
    mask = (c + base_count) < target
    below = plsc.all_reduce_population_count(mask)[0]
    c_above = jnp.max(jnp.where(mask, c + base_count, 0))
    return below, c_above


def _sc_kqv_body(rand_hbm, out_hbm, stage, hist, cand_a, cand_b, thr_buf,
                 sem0, sem1):
    wid = lax.axis_index("s") * _NC + lax.axis_index("c")
    base = wid * _ROWS_PER_W
    lane = lax.iota(jnp.int32, 16)
    ones16 = jnp.ones((16,), jnp.int32)
    zeros16 = jnp.zeros((16,), jnp.int32)

    def refine(shift, src, dst, cnt, j):
        def do_refine(cnt, j):
            hist[pl.ds(0, 16)] = zeros16
            nch = (cnt + 15) // 16

            def scat(i, _):
                v = src[pl.ds(i * 16, 16)]
                valid = lane < (cnt - i * 16)
                ku = plsc.bitcast(v + 1.0, jnp.int32)
                u = 15 - ((ku >> shift) & 15)
                plsc.addupdate_scatter(hist, [u], ones16, mask=valid)
                return 0
            lax.fori_loop(0, nch, scat, 0)

            c = plsc.cumsum(hist[pl.ds(0, 16)])
            below, c_above = _scan16(c, j, jnp.int32(0))
            sub = 15 - below
            j2 = j - c_above

            def comp(i, w):
                v = src[pl.ds(i * 16, 16)]
                valid = lane < (cnt - i * 16)
                ku = plsc.bitcast(v + 1.0, jnp.int32)
                u = (ku >> shift) & 15
                m = valid & (u == sub)
                plsc.store_compressed(dst.at[pl.ds(w, 16)], v, mask=m)
                return w + plsc.all_reduce_population_count(m)[0]
            cnt2 = lax.fori_loop(0, nch, comp, jnp.int32(0))
            return cnt2, j2

        def passthrough(cnt, j):
            dst[pl.ds(0, 16)] = src[pl.ds(0, 16)]
            return cnt, j

        return lax.cond(cnt > 16, do_refine, passthrough, cnt, j)

    def row_body(row_idx, buf, r):
        for i in range(16):
            hist[pl.ds(i * 16, 16)] = zeros16

        @plsc.parallel_loop(0, _CHUNKS, unroll=8)
        def _(i):
            v = stage[buf, r, pl.ds(i * 16, 16)]
            ku = plsc.bitcast(v + 1.0, jnp.int32)
            du = jnp.maximum((jnp.int32(0x3FFFFFFF) - ku) >> 15, 0)
            plsc.addupdate_scatter(hist, [du], ones16)

        def scan(i, carry):
            run, b_desc, c_above = carry
            c = plsc.cumsum(hist[pl.ds(i * 16, 16)])
            below, ca = _scan16(c, jnp.int32(K), run)
            return run + c[15], b_desc + below, jnp.maximum(c_above, ca)

        zero = jnp.int32(0)
        _, b_desc, c_above = lax.fori_loop(0, 16, scan, (zero, zero, zero),
                                           unroll=4)
        j = jnp.int32(K) - c_above

        def comp(i, w):
            v = stage[buf, r, pl.ds(i * 16, 16)]
            ku = plsc.bitcast(v + 1.0, jnp.int32)
            du = jnp.maximum((jnp.int32(0x3FFFFFFF) - ku) >> 15, 0)
            m = du == b_desc
            plsc.store_compressed(cand_a.at[pl.ds(w, 16)], v, mask=m)
            return w + plsc.all_reduce_population_count(m)[0]
        cnt = plsc.parallel_loop(0, _CHUNKS, unroll=8,
                                 carry=jnp.int32(0))(comp)

        cnt, j = refine(11, cand_a, cand_b, cnt, j)
        cnt, j = refine(7, cand_b, cand_a, cnt, j)

        v = cand_a[pl.ds(0, 16)]
        v = jnp.where(lane < cnt, v, -1.0)
        sv = plsc.sort_key_val(v, v, descending=True)[0]
        jc = jnp.minimum(j, jnp.minimum(cnt, 16))
        thr = jnp.max(jnp.where(lane == jc - 1, sv, -1.0))

        idx = jnp.full((16,), row_idx, jnp.int32)
        plsc.store_scatter(thr_buf, [idx, jnp.zeros((16,), jnp.int32)],
                           jnp.full((16,), thr, jnp.float32), mask=lane == 0)
        return 0

    sems = (sem0, sem1)

    def start_copy(b, buf):
        pltpu.async_copy(rand_hbm.at[pl.ds(base + b * _BLK, _BLK)],
                         stage.at[buf], sems[buf])

    def wait_copy(b, buf):
        pltpu.make_async_copy(rand_hbm.at[pl.ds(base + b * _BLK, _BLK)],
                              stage.at[buf], sems[buf]).wait()

    def process_blk(b, buf):
        def inner(r, _):
            return row_body(b * _BLK + r, buf, r)
        lax.fori_loop(0, _BLK, inner, 0)

    start_copy(0, 0)

    def blk_pair(p, _):
        b0 = p * 2
        start_copy(b0 + 1, 1)
        wait_copy(b0, 0)
        process_blk(b0, 0)

        @pl.when(b0 + 2 < _NBLK)
        def _():
            start_copy(b0 + 2, 0)

        wait_copy(b0 + 1, 1)
        process_blk(b0 + 1, 1)
        return 0

    lax.fori_loop(0, _NBLK // 2, blk_pair, 0)
    pltpu.sync_copy(thr_buf, out_hbm.at[pl.ds(base, _ROWS_PER_W)])


@functools.partial(
    pl.kernel,
    out_type=jax.ShapeDtypeStruct((BSZ, 1), jnp.float32),
    mesh=plsc.VectorSubcoreMesh(core_axis_name="c", subcore_axis_name="s"),
    scratch_types=[
        pltpu.VMEM((2, _BLK, W_KQV), jnp.float32),
        pltpu.VMEM((256,), jnp.int32),
        pltpu.VMEM((W_KQV + 16,), jnp.float32),
        pltpu.VMEM((W_KQV + 16,), jnp.float32),
        pltpu.VMEM((_ROWS_PER_W, 1), jnp.float32),
        pltpu.SemaphoreType.DMA,
        pltpu.SemaphoreType.DMA,
    ],
    compiler_params=pltpu.CompilerParams(needs_layout_passes=False),
)
def _sc_kqv_thresholds(rand_hbm, out_hbm, stage, hist, cand_a, cand_b,
                       thr_buf, sem0, sem1):
    _sc_kqv_body(rand_hbm, out_hbm, stage, hist, cand_a, cand_b,
                 thr_buf, sem0, sem1)


def kernel(rand_attn, noise_attn, rand_kqv, noise_k, noise_q, noise_v,
           perturb_attn, perturb_k, perturb_q, perturb_v):
    R = 256
    grid = (BSZ // R,)

    row_spec = pl.BlockSpec((R, TOTAL), lambda i: (i, 0))
    kqv_spec = pl.BlockSpec((R, W_KQV), lambda i: (i, 0))
    p_spec = pl.BlockSpec((1, TOTAL), lambda i: (0, 0))
    thr_spec = pl.BlockSpec((R, 1), lambda i: (i, 0))
    out3_spec = pl.BlockSpec((R, N_LAYERS, N_HEADS), lambda i: (i, 0, 0))
    out3_shape = jax.ShapeDtypeStruct((BSZ, N_LAYERS, N_HEADS), jnp.float32)

    attn_mask = pl.pallas_call(
        _attn_body,
        grid=grid,
        in_specs=[row_spec, row_spec, p_spec],
        out_specs=row_spec,
        out_shape=jax.ShapeDtypeStruct((BSZ, TOTAL), jnp.float32),
    )(rand_attn, noise_attn, perturb_attn.reshape(1, TOTAL))

    thr_kqv = _sc_kqv_thresholds(rand_kqv)

    k_mask, q_mask, v_mask = pl.pallas_call(
        _kqv_blend_body,
        grid=grid,
        in_specs=[kqv_spec, thr_spec, row_spec, row_spec, row_spec,
                  p_spec, p_spec, p_spec],
        out_specs=[row_spec, row_spec, row_spec],
        out_shape=[jax.ShapeDtypeStruct((BSZ, TOTAL), jnp.float32)] * 3,
    )(rand_kqv, thr_kqv, noise_k, noise_q, noise_v,
      perturb_k.reshape(1, TOTAL), perturb_q.reshape(1, TOTAL),
      perturb_v.reshape(1, TOTAL))

    shape = (BSZ, N_LAYERS, N_HEADS)
    return (attn_mask.reshape(shape), k_mask.reshape(shape),
            q_mask.reshape(shape), v_mask.reshape(shape))


# speedup vs baseline: 1.1112x; 1.0013x over previous
import functools

import jax
import jax.numpy as jnp
from jax import lax
from jax.experimental import pallas as pl
from jax.experimental.pallas import tpu as pltpu
from jax.experimental.pallas import tpu_sc as plsc

N_LAYERS = 32
N_HEADS = 32
BSZ = 4096
K = 256
TOTAL = N_LAYERS * N_HEADS
W_KQV = 3 * TOTAL

_SCALE = 16777216.0
_BITS = 24


def _row_threshold(m, k):
    rows = m.shape[0]
    lo = jnp.full((rows, 1), 0x3F800000, dtype=jnp.int32)
    hi = jnp.full((rows, 1), 0x40000000, dtype=jnp.int32)

    def body(_, carry):
        lo, hi = carry
        mid = (lo + hi) >> 1
        cnt = jnp.sum((m >= mid).astype(jnp.float32), axis=1, keepdims=True)
        pick = cnt >= float(k)
        lo = jnp.where(pick, mid, lo)
        hi = jnp.where(pick, hi, mid)
        return lo, hi

    lo, hi = jax.lax.fori_loop(0, 23, body, (lo, hi))
    return lo


def _attn_body(rand_ref, noise_ref, perturb_ref, out_ref):
    v = rand_ref[...]
    m = jax.lax.bitcast_convert_type(v + 1.0, jnp.int32)
    thr = _row_threshold(m, K)
    sel = m >= thr
    out_ref[...] = jnp.where(sel, noise_ref[...] + perturb_ref[...], 1.0)


def _kqv_blend_body(rand_ref, thr_ref, nk_ref, nq_ref, nv_ref,
                    pk_ref, pq_ref, pv_ref, ok_ref, oq_ref, ov_ref):
    v = rand_ref[...]
    sel = v >= thr_ref[...]
    ok_ref[...] = jnp.where(sel[:, 0:TOTAL], nk_ref[...] + pk_ref[...], 1.0)
    oq_ref[...] = jnp.where(sel[:, TOTAL:2 * TOTAL], nq_ref[...] + pq_ref[...], 1.0)
    ov_ref[...] = jnp.where(sel[:, 2 * TOTAL:3 * TOTAL], nv_ref[...] + pv_ref[...], 1.0)


_NC = 2
_NS = 16
_NW = _NC * _NS
_ROWS_PER_W = BSZ // _NW
_BLK = 8
_CHUNKS = W_KQV // 16


_NBLK = _ROWS_PER_W // _BLK
_KSCALE = 16777216.0


def _scan16(c, target, base_count):
    mask = (c + base_count) < target
    below = plsc.all_reduce_population_count(mask)[0]
    c_above = jnp.max(jnp.where(mask, c + base_count, 0))
    return below, c_above


def _sc_kqv_body(rand_hbm, out_hbm, stage, hist, cand_a, cand_b, thr_buf,
                 sem0, sem1):
    wid = lax.axis_index("s") * _NC + lax.axis_index("c")
    base = wid * _ROWS_PER_W
    lane = lax.iota(jnp.int32, 16)
    ones16 = jnp.ones((16,), jnp.int32)
    zeros16 = jnp.zeros((16,), jnp.int32)

    def refine(shift, src, dst, cnt, j):
        def do_refine(cnt, j):
            hist[pl.ds(0, 16)] = zeros16
            nch = (cnt + 15) // 16

            def scat(i, _):
                v = src[pl.ds(i * 16, 16)]
                valid = lane < (cnt - i * 16)
                ku = plsc.bitcast(v + 1.0, jnp.int32)
                u = 15 - ((ku >> shift) & 15)
                plsc.addupdate_scatter(hist, [u], ones16, mask=valid)
                return 0
            lax.fori_loop(0, nch, scat, 0)

            c = plsc.cumsum(hist[pl.ds(0, 16)])
            below, c_above = _scan16(c, j, jnp.int32(0))
            sub = 15 - below
            j2 = j - c_above

            def comp(i, w):
                v = src[pl.ds(i * 16, 16)]
                valid = lane < (cnt - i * 16)
                ku = plsc.bitcast(v + 1.0, jnp.int32)
                u = (ku >> shift) & 15
                m = valid & (u == sub)
                plsc.store_compressed(dst.at[pl.ds(w, 16)], v, mask=m)
                return w + plsc.all_reduce_population_count(m)[0]
            cnt2 = lax.fori_loop(0, nch, comp, jnp.int32(0))
            return cnt2, j2

        def passthrough(cnt, j):
            dst[pl.ds(0, 16)] = src[pl.ds(0, 16)]
            return cnt, j

        return lax.cond(cnt > 16, do_refine, passthrough, cnt, j)

    def row_body(row_idx, buf, r):
        for i in range(16):
            hist[pl.ds(i * 16, 16)] = zeros16

        @plsc.parallel_loop(0, _CHUNKS, unroll=8)
        def _(i):
            v = stage[buf, r, pl.ds(i * 16, 16)]
            ku = plsc.bitcast(v + 1.0, jnp.int32)
            du = jnp.maximum((jnp.int32(0x3FFFFFFF) - ku) >> 15, 0)
            plsc.addupdate_scatter(hist, [du], ones16)

        def scan(i, carry):
            run, b_desc, c_above = carry
            c = plsc.cumsum(hist[pl.ds(i * 16, 16)])
            below, ca = _scan16(c, jnp.int32(K), run)
            return run + c[15], b_desc + below, jnp.maximum(c_above, ca)

        zero = jnp.int32(0)
        _, b_desc, c_above = lax.fori_loop(0, 16, scan, (zero, zero, zero),
                                           unroll=4)
        j = jnp.int32(K) - c_above

        def comp(i, w):
            v = stage[buf, r, pl.ds(i * 16, 16)]
            ku = plsc.bitcast(v + 1.0, jnp.int32)
            du = jnp.maximum((jnp.int32(0x3FFFFFFF) - ku) >> 15, 0)
            m = du == b_desc
            plsc.store_compressed(cand_a.at[pl.ds(w, 16)], v, mask=m)
            return w + plsc.all_reduce_population_count(m)[0]
        cnt = plsc.parallel_loop(0, _CHUNKS, unroll=8,
                                 carry=jnp.int32(0))(comp)

        cnt, j = refine(11, cand_a, cand_b, cnt, j)
        cnt, j = refine(7, cand_b, cand_a, cnt, j)

        v = cand_a[pl.ds(0, 16)]
        v = jnp.where(lane < cnt, v, -1.0)
        sv = plsc.sort_key_val(v, v, descending=True)[0]
        jc = jnp.minimum(j, jnp.minimum(cnt, 16))
        thr = jnp.max(jnp.where(lane == jc - 1, sv, -1.0))

        idx = jnp.full((16,), row_idx, jnp.int32)
        plsc.store_scatter(thr_buf, [idx, jnp.zeros((16,), jnp.int32)],
                           jnp.full((16,), thr, jnp.float32), mask=lane == 0)
        return 0

    sems = (sem0, sem1)

    def start_copy(b, buf):
        pltpu.async_copy(rand_hbm.at[pl.ds(base + b * _BLK, _BLK)],
                         stage.at[buf], sems[buf])

    def wait_copy(b, buf):
        pltpu.make_async_copy(rand_hbm.at[pl.ds(base + b * _BLK, _BLK)],
                              stage.at[buf], sems[buf]).wait()

    def process_blk(b, buf):
        def inner(r, _):
            return row_body(b * _BLK + r, buf, r)
        lax.fori_loop(0, _BLK, inner, 0)

    start_copy(0, 0)

    def blk_pair(p, _):
        b0 = p * 2
        start_copy(b0 + 1, 1)
        wait_copy(b0, 0)
        process_blk(b0, 0)

        @pl.when(b0 + 2 < _NBLK)
        def _():
            start_copy(b0 + 2, 0)

        wait_copy(b0 + 1, 1)
        process_blk(b0 + 1, 1)
        return 0

    lax.fori_loop(0, _NBLK // 2, blk_pair, 0)
    pltpu.sync_copy(thr_buf, out_hbm.at[pl.ds(base, _ROWS_PER_W)])


@functools.partial(
    pl.kernel,
    out_type=jax.ShapeDtypeStruct((BSZ, 1), jnp.float32),
    mesh=plsc.VectorSubcoreMesh(core_axis_name="c", subcore_axis_name="s"),
    scratch_types=[
        pltpu.VMEM((2, _BLK, W_KQV), jnp.float32),
        pltpu.VMEM((256,), jnp.int32),
        pltpu.VMEM((W_KQV + 16,), jnp.float32),
        pltpu.VMEM((W_KQV + 16,), jnp.float32),
        pltpu.VMEM((_ROWS_PER_W, 1), jnp.float32),
        pltpu.SemaphoreType.DMA,
        pltpu.SemaphoreType.DMA,
    ],
    compiler_params=pltpu.CompilerParams(needs_layout_passes=False),
)
def _sc_kqv_thresholds(rand_hbm, out_hbm, stage, hist, cand_a, cand_b,
                       thr_buf, sem0, sem1):
    _sc_kqv_body(rand_hbm, out_hbm, stage, hist, cand_a, cand_b,
                 thr_buf, sem0, sem1)


def kernel(rand_attn, noise_attn, rand_kqv, noise_k, noise_q, noise_v,
           perturb_attn, perturb_k, perturb_q, perturb_v):
    R = 256
    grid = (BSZ // R,)

    row_spec = pl.BlockSpec((R, TOTAL), lambda i: (i, 0))
    kqv_spec = pl.BlockSpec((R, W_KQV), lambda i: (i, 0))
    p_spec = pl.BlockSpec((1, TOTAL), lambda i: (0, 0))
    thr_spec = pl.BlockSpec((R, 1), lambda i: (i, 0))
    out3_spec = pl.BlockSpec((R, N_LAYERS, N_HEADS), lambda i: (i, 0, 0))
    out3_shape = jax.ShapeDtypeStruct((BSZ, N_LAYERS, N_HEADS), jnp.float32)

    attn_mask = pl.pallas_call(
        _attn_body,
        grid=grid,
        in_specs=[row_spec, row_spec, p_spec],
        out_specs=row_spec,
        out_shape=jax.ShapeDtypeStruct((BSZ, TOTAL), jnp.float32),
    )(rand_attn, noise_attn, perturb_attn.reshape(1, TOTAL))

    thr_kqv = _sc_kqv_thresholds(rand_kqv)

    RB = 512
    b_row = pl.BlockSpec((RB, TOTAL), lambda i: (i, 0))
    b_kqv = pl.BlockSpec((RB, W_KQV), lambda i: (i, 0))
    b_thr = pl.BlockSpec((RB, 1), lambda i: (i, 0))
    k_mask, q_mask, v_mask = pl.pallas_call(
        _kqv_blend_body,
        grid=(BSZ // RB,),
        in_specs=[b_kqv, b_thr, b_row, b_row, b_row,
                  p_spec, p_spec, p_spec],
        out_specs=[b_row, b_row, b_row],
        out_shape=[jax.ShapeDtypeStruct((BSZ, TOTAL), jnp.float32)] * 3,
    )(rand_kqv, thr_kqv, noise_k, noise_q, noise_v,
      perturb_k.reshape(1, TOTAL), perturb_q.reshape(1, TOTAL),
      perturb_v.reshape(1, TOTAL))

    shape = (BSZ, N_LAYERS, N_HEADS)
    return (attn_mask.reshape(shape), k_mask.reshape(shape),
            q_mask.reshape(shape), v_mask.reshape(shape))
